# baseline (speedup 1.0000x reference)
import functools

import jax
import jax.numpy as jnp
from jax import lax
from jax.experimental import pallas as pl
from jax.experimental.pallas import tpu as pltpu
from jax.experimental.pallas import tpu_sc as plsc

B, L, V, D = 4096, 200, 1000000, 64
LANES = 16
LP = 208
NC, NS = 2, 16
NW = NC * NS
RPW = B // NW
HALF = LP // 2
NCHUNK = LP // LANES


def _dense_body(n64, i64, ub, w1, bb1, w2, bb2, pv, v_ref, s_ref):
    nh = lax.dot_general(n64[...], w1[...], (((1,), (1,)), ((), ())),
                         preferred_element_type=jnp.float32)
    ue = lax.dot_general(i64[...], w2[...], (((1,), (1,)), ((), ())),
                         preferred_element_type=jnp.float32)
    v_ref[...] = pv[0] * (nh + bb1[...][None, :]) + pv[1] * (ue + bb2[...][None, :])
    s_ref[...] = pv[2] * ub[...]


def _dense_stage(noise64, init64, ub, W1, b1, W2, b2, p234):
    return pl.pallas_call(
        _dense_body,
        out_shape=(
            jax.ShapeDtypeStruct((B, D), jnp.float32),
            jax.ShapeDtypeStruct((B,), jnp.float32),
        ),
        in_specs=[pl.BlockSpec(memory_space=pltpu.VMEM)] * 7
        + [pl.BlockSpec(memory_space=pltpu.SMEM)],
    )(noise64, init64, ub, W1, b1, W2, b2, p234)


def _sc_gather_dot(ids, ratings, table, item_bias, v, s, p15):
    mesh = plsc.VectorSubcoreMesh(core_axis_name="c", subcore_axis_name="s")

    @functools.partial(
        pl.kernel,
        mesh=mesh,
        compiler_params=pltpu.CompilerParams(
            needs_layout_passes=False, use_tc_tiling_on_sc=False),
        out_type=jax.ShapeDtypeStruct((B * L,), jnp.float32),
        scratch_types=[
            pltpu.VMEM((LP,), jnp.int32),
            pltpu.VMEM((LP,), jnp.int32),
            pltpu.VMEM((LP, D), jnp.float32),
            pltpu.VMEM((LP, D), jnp.float32),
            pltpu.VMEM((LP,), jnp.float32),
            pltpu.VMEM((LP,), jnp.float32),
            pltpu.VMEM((LP,), jnp.float32),
            pltpu.VMEM((LP,), jnp.float32),
            pltpu.VMEM((LP,), jnp.float32),
            pltpu.VMEM((LP,), jnp.float32),
            pltpu.VMEM((RPW * D,), jnp.float32),
            pltpu.VMEM((RPW + LANES,), jnp.float32),
            pltpu.VMEM((16,), jnp.float32),
            pltpu.SemaphoreType.DMA,
            pltpu.SemaphoreType.DMA,
            pltpu.SemaphoreType.DMA,
            pltpu.SemaphoreType.DMA,
            pltpu.SemaphoreType.DMA,
            pltpu.SemaphoreType.DMA,
        ],
    )
    def sc_kernel(ids_h, rat_h, tab_h, bias_h, v_h, s_h, p_h, out_h,
                  idx0, idx1, rows0, rows1, bias0, bias1, rat0, rat1,
                  outb0, outb1, v_loc, s_loc, p_loc,
                  sI0, sI1, sG0, sG1, sO0, sO1):
        idx = (idx0, idx1)
        rows = (rows0, rows1)
        biasb = (bias0, bias1)
        ratb = (rat0, rat1)
        outb = (outb0, outb1)
        sI = (sI0, sI1)
        sG = (sG0, sG1)
        sO = (sO0, sO1)

        wid = lax.axis_index("s") * NC + lax.axis_index("c")
        base = wid * RPW

        pltpu.sync_copy(v_h.at[pl.ds(base * D, RPW * D)], v_loc)
        pltpu.sync_copy(s_h.at[pl.ds(base, RPW)], s_loc.at[pl.ds(0, RPW)])
        pltpu.sync_copy(p_h, p_loc)
        zeros16 = jnp.zeros((LANES,), jnp.int32)
        for ph in range(2):
            idx[ph][pl.ds(LP - LANES, LANES)] = zeros16

        def idx_copy(r, ph):
            return pltpu.make_async_copy(
                ids_h.at[pl.ds((base + r) * L, L)], idx[ph].at[pl.ds(0, L)],
                sI[ph])

        def gather_copies(r, ph):
            return (
                pltpu.make_async_copy(
                    tab_h.at[idx[ph].at[pl.ds(0, HALF)]],
                    rows[ph].at[pl.ds(0, HALF)], sG[ph]),
                pltpu.make_async_copy(
                    tab_h.at[idx[ph].at[pl.ds(HALF, HALF)]],
                    rows[ph].at[pl.ds(HALF, HALF)], sG[ph]),
                pltpu.make_async_copy(
                    bias_h.at[idx[ph].at[pl.ds(0, HALF)]],
                    biasb[ph].at[pl.ds(0, HALF)], sG[ph]),
                pltpu.make_async_copy(
                    bias_h.at[idx[ph].at[pl.ds(HALF, HALF)]],
                    biasb[ph].at[pl.ds(HALF, HALF)], sG[ph]),
                pltpu.make_async_copy(
                    rat_h.at[pl.ds((base + r) * L, L)],
                    ratb[ph].at[pl.ds(0, L)], sG[ph]),
            )

        def out_copy(r, ph):
            return pltpu.make_async_copy(
                outb[ph].at[pl.ds(0, L)], out_h.at[pl.ds((base + r) * L, L)],
                sO[ph])

        lane_iota = lax.iota(jnp.int32, LANES)

        def compute(r, ph):
            s_val = s_loc[pl.ds(r, LANES)][0]
            pvec = p_loc[...]
            p1v = pvec[0]
            p5v = pvec[1]
            vrow = [v_loc[pl.ds(r * D + d0, LANES)]
                    for d0 in range(0, D, LANES)]

            def chunk(c, carry):
                j0 = c * LANES
                res = (p1v * ratb[ph][pl.ds(j0, LANES)]
                       + p5v * biasb[ph][pl.ds(j0, LANES)] + s_val)
                for jj in range(LANES):
                    j = j0 + jj
                    t = rows[ph][j, pl.ds(0, LANES)] * vrow[0]
                    for q in range(1, D // LANES):
                        t = t + rows[ph][j, pl.ds(q * LANES, LANES)] * vrow[q]
                    dotv = jnp.sum(t)
                    res = jnp.where(lane_iota == jj, res + dotv, res)
                outb[ph][pl.ds(j0, LANES)] = res
                return carry

            lax.fori_loop(0, NCHUNK, chunk, 0)

        idx_copy(0, 0).start()
        idx_copy(0, 0).wait()
        for c in gather_copies(0, 0):
            c.start()
        idx_copy(1, 1).start()

        def step(i, carry):
            for ph in range(2):
                r = 2 * i + ph
                for c in gather_copies(r, ph):
                    c.wait()

                @pl.when(r + 1 < RPW)
                def _():
                    idx_copy(r + 1, 1 - ph).wait()
                    for c in gather_copies(r + 1, 1 - ph):
                        c.start()

                @pl.when(r + 2 < RPW)
                def _():
                    idx_copy(r + 2, ph).start()

                @pl.when(r >= 2)
                def _():
                    out_copy(r - 2, ph).wait()

                compute(r, ph)
                out_copy(r, ph).start()
            return carry

        lax.fori_loop(0, RPW // 2, step, 0)
        out_copy(RPW - 2, 0).wait()
        out_copy(RPW - 1, 1).wait()

    return sc_kernel(ids, ratings, table, item_bias, v, s, p15)


def kernel(ratings, item_ids, noise, init_user_emb, table, W1, b1, W2, b2,
           item_bias, p1, p2, p3, p4, p5):
    noise64 = noise[:, :D]
    init64 = init_user_emb[:, :D]
    ub = init_user_emb[:, D]
    ids = item_ids.astype(jnp.int32)
    p234 = jnp.concatenate([p2, p3, p4]).astype(jnp.float32)
    p15 = jnp.zeros((16,), jnp.float32).at[0].set(p1[0]).at[1].set(p5[0])

    v, s = _dense_stage(noise64, init64, ub, W1, b1, W2, b2, p234)
    out = _sc_gather_dot(ids.reshape(-1), ratings.reshape(-1), table,
                         item_bias, v.reshape(-1), s, p15)
    return out.reshape(B, L)
